# baseline (speedup 1.0000x reference)
import functools
import math

import jax
import jax.numpy as jnp
from jax import lax
from jax.experimental import pallas as pl
from jax.experimental.pallas import tpu as pltpu
from jax.experimental.pallas import tpu_sc as plsc

EMBED = 64
SCALE = math.sqrt(EMBED)
CH = 128
NC = 2
NS = 16
NW = NC * NS


def _make_sc_kernel(n_chunks: int):
  per_w = n_chunks * CH
  mesh = plsc.VectorSubcoreMesh(core_axis_name="c", subcore_axis_name="s")

  @functools.partial(
      pl.kernel,
      out_type=jax.ShapeDtypeStruct((NW * per_w, EMBED), jnp.float32),
      mesh=mesh,
      compiler_params=pltpu.CompilerParams(use_tc_tiling_on_sc=False),
      scratch_types=[
          pltpu.VMEM((n_chunks, CH), jnp.int32),
          pltpu.VMEM((CH, EMBED), jnp.float32),
          pltpu.VMEM((CH, EMBED), jnp.float32),
          pltpu.SemaphoreType.DMA,
          pltpu.SemaphoreType.DMA,
      ],
  )
  def k(base_hbm, tab_hbm, idx_hbm, out_hbm, idx_v, rows_a, rows_b,
        sem_a, sem_b):
    wid = lax.axis_index("s") * NC + lax.axis_index("c")
    base = wid * per_w
    pltpu.sync_copy(idx_hbm.at[wid], idx_v)

    @pl.loop(0, n_chunks)
    def _chunk(g):
      ida = idx_v.at[g]
      ca = pltpu.async_copy(base_hbm.at[ida], rows_a, sem_a)
      cb = pltpu.async_copy(tab_hbm.at[ida], rows_b, sem_b)
      ca.wait()
      cb.wait()

      @pl.loop(0, CH)
      def _row(j):
        for d in range(EMBED // 16):
          sl = pl.ds(d * 16, 16)
          rows_a[j, sl] = rows_a[j, sl] + SCALE * rows_b[j, sl]

      pltpu.sync_copy(rows_a, out_hbm.at[pl.ds(base + g * CH, CH)])

  return k


def kernel(tokens, base_table, table):
  shape = tokens.shape
  n = tokens.size
  idx = tokens.reshape(-1).astype(jnp.int32)
  n_chunks = -(-n // (NW * CH))
  n_pad = n_chunks * CH * NW
  if n_pad != n:
    idx = jnp.pad(idx, (0, n_pad - n))
  idx3 = idx.reshape(NW, n_chunks, CH)
  out = _make_sc_kernel(n_chunks)(base_table, table, idx3)
  if n_pad != n:
    out = out[:n]
  return out.reshape(*shape, EMBED)
